# Initial kernel scaffold; baseline (speedup 1.0000x reference)
import jax
import jax.numpy as jnp
from jax.experimental import pallas as pl


def kernel(x, edge_index, W_l1, b_l1, W_r1, W_l2, b_l2, W_r2):
    raise NotImplementedError("write your pallas kernel here")


# speedup vs baseline: 1.5555x; 1.5555x over previous
import jax
import jax.numpy as jnp
from jax import lax
from jax.experimental import pallas as pl
from jax.experimental.pallas import tpu as pltpu
from jax.experimental.pallas import tpu_sc as plsc

N = 10000
E = 320000
D = 128
NC = 2
NS = 16
NW = NC * NS
G = 16
NG = E // G
NPW = NG // NW
NP = 10240
RPT = NP // NS
ZCH = 128


def _mesh():
    return plsc.VectorSubcoreMesh(core_axis_name="c", subcore_axis_name="s",
                                  num_cores=NC, num_subcores=NS)


def _sc_cnt_body(dst_hbm, zcnt_hbm, ones_hbm, cnt_out, idx_d, ones_v, zcnt_v,
                 cnt_sh):
    c = lax.axis_index("c")
    s = lax.axis_index("s")
    wid = s * NC + c

    pltpu.sync_copy(zcnt_hbm, zcnt_v)
    for j in range(RPT // ZCH):
        pltpu.sync_copy(zcnt_v, cnt_sh.at[pl.ds(s * RPT + j * ZCH, ZCH)])
    pltpu.sync_copy(ones_hbm, ones_v)
    plsc.subcore_barrier()

    def it(i, carry):
        base = (wid + NW * i) * G
        pltpu.sync_copy(dst_hbm.at[pl.ds(base, G)], idx_d)
        pltpu.sync_copy(ones_v, cnt_sh.at[idx_d], add=True)
        return carry

    lax.fori_loop(0, NPW, it, 0)
    plsc.subcore_barrier()

    out_base = c * NP + s * RPT
    for j in range(RPT // ZCH):
        pltpu.sync_copy(cnt_sh.at[pl.ds(s * RPT + j * ZCH, ZCH)], zcnt_v)
        pltpu.sync_copy(zcnt_v, cnt_out.at[pl.ds(out_base + j * ZCH, ZCH)])


def _sc_counts(dst):
    zcnt = jnp.zeros((ZCH, D), jnp.float32)
    ones16 = jnp.ones((G, D), jnp.float32)
    k = pl.kernel(
        _sc_cnt_body,
        out_type=jax.ShapeDtypeStruct((NC * NP, D), jnp.float32),
        mesh=_mesh(),
        scratch_types=[
            pltpu.VMEM((G,), jnp.int32),
            pltpu.VMEM((G, D), jnp.float32),
            pltpu.VMEM((ZCH, D), jnp.float32),
            pltpu.VMEM_SHARED((NP, D), jnp.float32),
        ],
    )
    return k(dst, zcnt, ones16)


def _sc_agg_body(x_hbm, src_hbm, dst_hbm, zrows_hbm, acc_out, idx_s, idx_d,
                 rows, rows16, sem, acc_sh):
    c = lax.axis_index("c")
    s = lax.axis_index("s")
    wid = s * NC + c

    pltpu.sync_copy(zrows_hbm, rows)
    for j in range(RPT // ZCH):
        pltpu.sync_copy(rows, acc_sh.at[pl.ds(s * RPT + j * ZCH, ZCH)])
    plsc.subcore_barrier()

    def it(i, carry):
        base = (wid + NW * i) * G
        pltpu.sync_copy(src_hbm.at[pl.ds(base, G)], idx_s)
        pltpu.sync_copy(dst_hbm.at[pl.ds(base, G)], idx_d)
        pltpu.async_copy(x_hbm.at[idx_s], rows16, sem).wait()
        pltpu.sync_copy(rows16, acc_sh.at[idx_d], add=True)
        return carry

    lax.fori_loop(0, NPW, it, 0)
    plsc.subcore_barrier()

    out_base = c * NP + s * RPT
    for j in range(RPT // ZCH):
        pltpu.sync_copy(acc_sh.at[pl.ds(s * RPT + j * ZCH, ZCH)], rows)
        pltpu.sync_copy(rows, acc_out.at[pl.ds(out_base + j * ZCH, ZCH)])


def _sc_aggregate(x, src, dst):
    zrows = jnp.zeros((ZCH, D), jnp.float32)
    k = pl.kernel(
        _sc_agg_body,
        out_type=jax.ShapeDtypeStruct((NC * NP, D), jnp.float32),
        mesh=_mesh(),
        scratch_types=[
            pltpu.VMEM((G,), jnp.int32),
            pltpu.VMEM((G,), jnp.int32),
            pltpu.VMEM((ZCH, D), jnp.float32),
            pltpu.VMEM((G, D), jnp.float32),
            pltpu.SemaphoreType.DMA,
            pltpu.VMEM_SHARED((NP, D), jnp.float32),
        ],
    )
    return k(x, src, dst, zrows)


def _tc_combine_body(acca, accb, cnta, cntb, x_ref, wl, wr, b_ref, o_ref):
    cnt = cnta[:, 0:1] + cntb[:, 0:1]
    inv = 1.0 / jnp.maximum(cnt, 1.0)
    agg = (acca[...] + accb[...]) * inv
    h = jnp.dot(agg, wl[...], preferred_element_type=jnp.float32)
    h = h + jnp.dot(x_ref[...], wr[...], preferred_element_type=jnp.float32)
    o_ref[...] = jnp.maximum(h + b_ref[...], 0.0)


def _tc_combine(acc, cnt, x, W_l, W_r, b):
    BR = 1000
    grid = (N // BR,)
    row_spec = pl.BlockSpec((BR, D), lambda i: (i, 0))
    cnt_spec = pl.BlockSpec((BR, D), lambda i: (i, 0))
    full_spec = pl.BlockSpec((D, D), lambda i: (0, 0))
    return pl.pallas_call(
        _tc_combine_body,
        grid=grid,
        in_specs=[row_spec, row_spec, cnt_spec, cnt_spec, row_spec,
                  full_spec, full_spec, pl.BlockSpec((1, D), lambda i: (0, 0))],
        out_specs=row_spec,
        out_shape=jax.ShapeDtypeStruct((N, D), jnp.float32),
    )(acc[:N], acc[NP:NP + N], cnt[:N], cnt[NP:NP + N], x, W_l, W_r, b.reshape(1, D))


def kernel(x, edge_index, W_l1, b_l1, W_r1, W_l2, b_l2, W_r2):
    src = edge_index[0].astype(jnp.int32)
    dst = edge_index[1].astype(jnp.int32)
    cnt = _sc_counts(dst)
    acc1 = _sc_aggregate(x, src, dst)
    h = _tc_combine(acc1, cnt, x, W_l1, W_r1, b_l1)
    acc2 = _sc_aggregate(h, src, dst)
    return _tc_combine(acc2, cnt, h, W_l2, W_r2, b_l2)


# speedup vs baseline: 5.6579x; 3.6374x over previous
import jax
import jax.numpy as jnp
from jax import lax
from jax.experimental import pallas as pl
from jax.experimental.pallas import tpu as pltpu
from jax.experimental.pallas import tpu_sc as plsc

N = 10000
E = 320000
D = 128
NC = 2
NS = 16
NW = NC * NS
G = 16
CH = 128
NCH = E // CH
NP = 10240
RPT = NP // NS
ZCH = 128


def _mesh():
    return plsc.VectorSubcoreMesh(core_axis_name="c", subcore_axis_name="s",
                                  num_cores=NC, num_subcores=NS)


def _sc_cnt_body(dst_hbm, zcnt_hbm, ones_hbm, cnt_out, idx_d, ones_v, zcnt_v,
                 cnt_sh):
    c = lax.axis_index("c")
    s = lax.axis_index("s")
    wid = s * NC + c

    pltpu.sync_copy(zcnt_hbm, zcnt_v)
    for j in range(RPT // ZCH):
        pltpu.sync_copy(zcnt_v, cnt_sh.at[pl.ds(s * RPT + j * ZCH, ZCH)])
    pltpu.sync_copy(ones_hbm, ones_v)
    plsc.subcore_barrier()

    n_i = (NCH - wid + NW - 1) // NW

    def it(i, carry):
        base = (wid + NW * i) * CH
        pltpu.sync_copy(dst_hbm.at[pl.ds(base, CH)], idx_d)
        pltpu.sync_copy(ones_v, cnt_sh.at[idx_d], add=True)
        return carry

    lax.fori_loop(0, n_i, it, 0)
    plsc.subcore_barrier()

    out_base = c * NP + s * RPT
    for j in range(RPT // ZCH):
        pltpu.sync_copy(cnt_sh.at[pl.ds(s * RPT + j * ZCH, ZCH)], zcnt_v)
        pltpu.sync_copy(zcnt_v, cnt_out.at[pl.ds(out_base + j * ZCH, ZCH)])


def _sc_counts(dst):
    zcnt = jnp.zeros((ZCH, D), jnp.float32)
    ones16 = jnp.ones((CH, D), jnp.float32)
    k = pl.kernel(
        _sc_cnt_body,
        out_type=jax.ShapeDtypeStruct((NC * NP, D), jnp.float32),
        mesh=_mesh(),
        scratch_types=[
            pltpu.VMEM((CH,), jnp.int32),
            pltpu.VMEM((CH, D), jnp.float32),
            pltpu.VMEM((ZCH, D), jnp.float32),
            pltpu.VMEM_SHARED((NP, D), jnp.float32),
        ],
    )
    return k(dst, zcnt, ones16)


def _sc_agg_body(x_hbm, src_hbm, dst_hbm, zrows_hbm, acc_out, idx_s, idx_d,
                 rows, rows16, sem, acc_sh):
    c = lax.axis_index("c")
    s = lax.axis_index("s")
    wid = s * NC + c

    pltpu.sync_copy(zrows_hbm, rows)
    for j in range(RPT // ZCH):
        pltpu.sync_copy(rows, acc_sh.at[pl.ds(s * RPT + j * ZCH, ZCH)])
    plsc.subcore_barrier()

    n_i = (NCH - wid + NW - 1) // NW

    def it(i, carry):
        base = (wid + NW * i) * CH
        pltpu.sync_copy(src_hbm.at[pl.ds(base, CH)], idx_s)
        pltpu.sync_copy(dst_hbm.at[pl.ds(base, CH)], idx_d)
        pltpu.async_copy(x_hbm.at[idx_s], rows16, sem).wait()
        pltpu.sync_copy(rows16, acc_sh.at[idx_d], add=True)
        return carry

    lax.fori_loop(0, n_i, it, 0)
    plsc.subcore_barrier()

    out_base = c * NP + s * RPT
    for j in range(RPT // ZCH):
        pltpu.sync_copy(acc_sh.at[pl.ds(s * RPT + j * ZCH, ZCH)], rows)
        pltpu.sync_copy(rows, acc_out.at[pl.ds(out_base + j * ZCH, ZCH)])


def _sc_aggregate(x, src, dst):
    zrows = jnp.zeros((ZCH, D), jnp.float32)
    k = pl.kernel(
        _sc_agg_body,
        out_type=jax.ShapeDtypeStruct((NC * NP, D), jnp.float32),
        mesh=_mesh(),
        scratch_types=[
            pltpu.VMEM((CH,), jnp.int32),
            pltpu.VMEM((CH,), jnp.int32),
            pltpu.VMEM((ZCH, D), jnp.float32),
            pltpu.VMEM((CH, D), jnp.float32),
            pltpu.SemaphoreType.DMA,
            pltpu.VMEM_SHARED((NP, D), jnp.float32),
        ],
    )
    return k(x, src, dst, zrows)


def _tc_combine_body(acca, accb, cnta, cntb, x_ref, wl, wr, b_ref, o_ref):
    cnt = cnta[:, 0:1] + cntb[:, 0:1]
    inv = 1.0 / jnp.maximum(cnt, 1.0)
    agg = (acca[...] + accb[...]) * inv
    h = jnp.dot(agg, wl[...], preferred_element_type=jnp.float32)
    h = h + jnp.dot(x_ref[...], wr[...], preferred_element_type=jnp.float32)
    o_ref[...] = jnp.maximum(h + b_ref[...], 0.0)


def _tc_combine(acc, cnt, x, W_l, W_r, b):
    BR = 1000
    grid = (N // BR,)
    row_spec = pl.BlockSpec((BR, D), lambda i: (i, 0))
    cnt_spec = pl.BlockSpec((BR, D), lambda i: (i, 0))
    full_spec = pl.BlockSpec((D, D), lambda i: (0, 0))
    return pl.pallas_call(
        _tc_combine_body,
        grid=grid,
        in_specs=[row_spec, row_spec, cnt_spec, cnt_spec, row_spec,
                  full_spec, full_spec, pl.BlockSpec((1, D), lambda i: (0, 0))],
        out_specs=row_spec,
        out_shape=jax.ShapeDtypeStruct((N, D), jnp.float32),
    )(acc[:N], acc[NP:NP + N], cnt[:N], cnt[NP:NP + N], x, W_l, W_r, b.reshape(1, D))


def kernel(x, edge_index, W_l1, b_l1, W_r1, W_l2, b_l2, W_r2):
    src = edge_index[0].astype(jnp.int32)
    dst = edge_index[1].astype(jnp.int32)
    cnt = _sc_counts(dst)
    acc1 = _sc_aggregate(x, src, dst)
    h = _tc_combine(acc1, cnt, x, W_l1, W_r1, b_l1)
    acc2 = _sc_aggregate(h, src, dst)
    return _tc_combine(acc2, cnt, h, W_l2, W_r2, b_l2)


# speedup vs baseline: 7.5947x; 1.3423x over previous
import jax
import jax.numpy as jnp
from jax import lax
from jax.experimental import pallas as pl
from jax.experimental.pallas import tpu as pltpu
from jax.experimental.pallas import tpu_sc as plsc

N = 10000
E = 320000
D = 128
NC = 2
NS = 16
NW = NC * NS
G = 16
CH = 128
NCH = E // CH
NP = 10240
RPT = NP // NS
ZCH = 128


def _mesh():
    return plsc.VectorSubcoreMesh(core_axis_name="c", subcore_axis_name="s",
                                  num_cores=NC, num_subcores=NS)


def _sc_cnt_body(dst_hbm, zcnt_hbm, ones_hbm, cnt_out, idx_d, idx_b, ones_v,
                 zcnt_v, sem_a, sem_b, cnt_sh):
    c = lax.axis_index("c")
    s = lax.axis_index("s")
    wid = s * NC + c

    pltpu.sync_copy(zcnt_hbm, zcnt_v)
    for j in range(RPT // ZCH):
        pltpu.sync_copy(zcnt_v, cnt_sh.at[pl.ds(s * RPT + j * ZCH, ZCH)])
    pltpu.sync_copy(ones_hbm, ones_v)
    plsc.subcore_barrier()

    n_i = (NCH - wid + NW - 1) // NW

    def itp(j, carry):
        base_a = (wid + NW * 2 * j) * CH
        base_b = (wid + NW * (2 * j + 1)) * CH
        da = pltpu.async_copy(dst_hbm.at[pl.ds(base_a, CH)], idx_d, sem_a)
        db = pltpu.async_copy(dst_hbm.at[pl.ds(base_b, CH)], idx_b, sem_b)
        da.wait()
        pltpu.sync_copy(ones_v, cnt_sh.at[idx_d], add=True)
        db.wait()
        pltpu.sync_copy(ones_v, cnt_sh.at[idx_b], add=True)
        return carry

    lax.fori_loop(0, n_i // 2, itp, 0)

    @pl.when(n_i % 2 == 1)
    def _tail():
        base = (wid + NW * (n_i - 1)) * CH
        pltpu.sync_copy(dst_hbm.at[pl.ds(base, CH)], idx_d)
        pltpu.sync_copy(ones_v, cnt_sh.at[idx_d], add=True)

    plsc.subcore_barrier()

    out_base = c * NP + s * RPT
    for j in range(RPT // ZCH):
        pltpu.sync_copy(cnt_sh.at[pl.ds(s * RPT + j * ZCH, ZCH)], zcnt_v)
        pltpu.sync_copy(zcnt_v, cnt_out.at[pl.ds(out_base + j * ZCH, ZCH)])


def _sc_counts(dst):
    zcnt = jnp.zeros((ZCH, D), jnp.float32)
    ones16 = jnp.ones((CH, D), jnp.float32)
    k = pl.kernel(
        _sc_cnt_body,
        out_type=jax.ShapeDtypeStruct((NC * NP, D), jnp.float32),
        mesh=_mesh(),
        scratch_types=[
            pltpu.VMEM((CH,), jnp.int32),
            pltpu.VMEM((CH,), jnp.int32),
            pltpu.VMEM((CH, D), jnp.float32),
            pltpu.VMEM((ZCH, D), jnp.float32),
            pltpu.SemaphoreType.DMA,
            pltpu.SemaphoreType.DMA,
            pltpu.VMEM_SHARED((NP, D), jnp.float32),
        ],
    )
    return k(dst, zcnt, ones16)


def _sc_agg_body(x_hbm, src_hbm, dst_hbm, zrows_hbm, acc_out, idx_s, idx_d,
                 idx_bs, idx_bd, rows, rows_b, sem_ia, sem_ja, sem_ib,
                 sem_jb, sem_ga, sem_gb, acc_sh):
    c = lax.axis_index("c")
    s = lax.axis_index("s")
    wid = s * NC + c

    pltpu.sync_copy(zrows_hbm, rows)
    for j in range(RPT // ZCH):
        pltpu.sync_copy(rows, acc_sh.at[pl.ds(s * RPT + j * ZCH, ZCH)])
    plsc.subcore_barrier()

    n_i = (NCH - wid + NW - 1) // NW

    def itp(j, carry):
        base_a = (wid + NW * 2 * j) * CH
        base_b = (wid + NW * (2 * j + 1)) * CH
        das = pltpu.async_copy(src_hbm.at[pl.ds(base_a, CH)], idx_s, sem_ia)
        dad = pltpu.async_copy(dst_hbm.at[pl.ds(base_a, CH)], idx_d, sem_ja)
        dbs = pltpu.async_copy(src_hbm.at[pl.ds(base_b, CH)], idx_bs, sem_ib)
        dbd = pltpu.async_copy(dst_hbm.at[pl.ds(base_b, CH)], idx_bd, sem_jb)
        das.wait()
        ga = pltpu.async_copy(x_hbm.at[idx_s], rows, sem_ga)
        dbs.wait()
        gb = pltpu.async_copy(x_hbm.at[idx_bs], rows_b, sem_gb)
        ga.wait()
        dad.wait()
        pltpu.sync_copy(rows, acc_sh.at[idx_d], add=True)
        gb.wait()
        dbd.wait()
        pltpu.sync_copy(rows_b, acc_sh.at[idx_bd], add=True)
        return carry

    lax.fori_loop(0, n_i // 2, itp, 0)

    @pl.when(n_i % 2 == 1)
    def _tail():
        base = (wid + NW * (n_i - 1)) * CH
        pltpu.sync_copy(src_hbm.at[pl.ds(base, CH)], idx_s)
        pltpu.sync_copy(dst_hbm.at[pl.ds(base, CH)], idx_d)
        pltpu.async_copy(x_hbm.at[idx_s], rows, sem_ga).wait()
        pltpu.sync_copy(rows, acc_sh.at[idx_d], add=True)

    plsc.subcore_barrier()

    out_base = c * NP + s * RPT
    for j in range(RPT // ZCH):
        pltpu.sync_copy(acc_sh.at[pl.ds(s * RPT + j * ZCH, ZCH)], rows)
        pltpu.sync_copy(rows, acc_out.at[pl.ds(out_base + j * ZCH, ZCH)])


def _sc_aggregate(x, src, dst):
    zrows = jnp.zeros((ZCH, D), jnp.float32)
    k = pl.kernel(
        _sc_agg_body,
        out_type=jax.ShapeDtypeStruct((NC * NP, D), jnp.float32),
        mesh=_mesh(),
        scratch_types=[
            pltpu.VMEM((CH,), jnp.int32),
            pltpu.VMEM((CH,), jnp.int32),
            pltpu.VMEM((CH,), jnp.int32),
            pltpu.VMEM((CH,), jnp.int32),
            pltpu.VMEM((ZCH, D), jnp.float32),
            pltpu.VMEM((CH, D), jnp.float32),
            pltpu.SemaphoreType.DMA,
            pltpu.SemaphoreType.DMA,
            pltpu.SemaphoreType.DMA,
            pltpu.SemaphoreType.DMA,
            pltpu.SemaphoreType.DMA,
            pltpu.SemaphoreType.DMA,
            pltpu.VMEM_SHARED((NP, D), jnp.float32),
        ],
    )
    return k(x, src, dst, zrows)


def _tc_combine_body(acca, accb, cnta, cntb, x_ref, wl, wr, b_ref, o_ref):
    cnt = cnta[:, 0:1] + cntb[:, 0:1]
    inv = 1.0 / jnp.maximum(cnt, 1.0)
    agg = (acca[...] + accb[...]) * inv
    h = jnp.dot(agg, wl[...], preferred_element_type=jnp.float32)
    h = h + jnp.dot(x_ref[...], wr[...], preferred_element_type=jnp.float32)
    o_ref[...] = jnp.maximum(h + b_ref[...], 0.0)


def _tc_combine(acc, cnt, x, W_l, W_r, b):
    BR = 1000
    grid = (N // BR,)
    row_spec = pl.BlockSpec((BR, D), lambda i: (i, 0))
    cnt_spec = pl.BlockSpec((BR, D), lambda i: (i, 0))
    full_spec = pl.BlockSpec((D, D), lambda i: (0, 0))
    return pl.pallas_call(
        _tc_combine_body,
        grid=grid,
        in_specs=[row_spec, row_spec, cnt_spec, cnt_spec, row_spec,
                  full_spec, full_spec, pl.BlockSpec((1, D), lambda i: (0, 0))],
        out_specs=row_spec,
        out_shape=jax.ShapeDtypeStruct((N, D), jnp.float32),
    )(acc[:N], acc[NP:NP + N], cnt[:N], cnt[NP:NP + N], x, W_l, W_r, b.reshape(1, D))


def kernel(x, edge_index, W_l1, b_l1, W_r1, W_l2, b_l2, W_r2):
    src = edge_index[0].astype(jnp.int32)
    dst = edge_index[1].astype(jnp.int32)
    cnt = _sc_counts(dst)
    acc1 = _sc_aggregate(x, src, dst)
    h = _tc_combine(acc1, cnt, x, W_l1, W_r1, b_l1)
    acc2 = _sc_aggregate(h, src, dst)
    return _tc_combine(acc2, cnt, h, W_l2, W_r2, b_l2)
